# baseline (speedup 1.0000x reference)
import functools
import math

import jax
import jax.numpy as jnp
from jax import lax
from jax.experimental import pallas as pl
from jax.experimental.pallas import tpu as pltpu
from jax.experimental.pallas import tpu_sc as plsc

_D = 128
_NC = 2
_NS = 16
_NW = _NC * _NS
_B = 128
_RING = 2


def _pad_to(n, m):
    return ((n + m - 1) // m) * m


@functools.cache
def _make_deg_kernel(EP, NP):
    nb = EP // (_NW * _B)
    rpt = NP // _NS
    mesh = plsc.VectorSubcoreMesh(core_axis_name="c", subcore_axis_name="s",
                                  num_cores=_NC, num_subcores=_NS)

    @functools.partial(
        pl.kernel,
        out_type=jax.ShapeDtypeStruct((_NC, NP), jnp.float32),
        mesh=mesh,
        scratch_types=[
            pltpu.VMEM((nb, _B), jnp.int32),
            pltpu.VMEM((_B,), jnp.float32),
            pltpu.VMEM((rpt,), jnp.float32),
            pltpu.VMEM_SHARED((NP,), jnp.float32),
        ],
    )
    def deg_kernel(dst_hbm, out_hbm, idx_v, ones_v, zeros_v, hist_s):
        cid = lax.axis_index("c")
        sid = lax.axis_index("s")
        wid = sid * _NC + cid
        pltpu.sync_copy(dst_hbm.at[wid], idx_v)
        one16 = jnp.ones((16,), jnp.float32)
        zero16 = jnp.zeros((16,), jnp.float32)
        for i in range(_B // 16):
            ones_v[pl.ds(i * 16, 16)] = one16

        def zbody(i, c):
            zeros_v[pl.ds(pl.multiple_of(i * 16, 8), 16)] = zero16
            return c
        lax.fori_loop(0, rpt // 16, zbody, 0)
        r0 = pl.multiple_of(sid * rpt, 8)
        pltpu.sync_copy(zeros_v, hist_s.at[pl.ds(r0, rpt)])
        plsc.subcore_barrier()

        def body(j, c):
            pltpu.sync_copy(ones_v, hist_s.at[idx_v.at[j]], add=True)
            return c
        lax.fori_loop(0, nb, body, 0)
        plsc.subcore_barrier()
        pltpu.sync_copy(hist_s.at[pl.ds(r0, rpt)], out_hbm.at[cid, pl.ds(r0, rpt)])

    return deg_kernel


@functools.cache
def _make_scatter_kernel(EP, NP, B=_B, RING=_RING):
    nb = EP // (_NW * B)
    rpt = NP // _NS
    assert nb % RING == 0
    mesh = plsc.VectorSubcoreMesh(core_axis_name="c", subcore_axis_name="s",
                                  num_cores=_NC, num_subcores=_NS)

    @functools.partial(
        pl.kernel,
        out_type=jax.ShapeDtypeStruct((_NC, NP, _D), jnp.float32),
        mesh=mesh,
        scratch_types=(
            [pltpu.VMEM((nb, B), jnp.int32),
             pltpu.VMEM((2 * RING, B), jnp.int32),
             pltpu.VMEM((RING, B, _D), jnp.float32),
             pltpu.VMEM_SHARED((NP, _D), jnp.float32)]
            + [pltpu.SemaphoreType.DMA for _ in range(RING)]
        ),
    )
    def scatter_kernel(g_hbm, packed_hbm, zeros_hbm, out_hbm, idx_v, sd_v,
                       rows_v, acc_s, *sems):
        srcb = [sd_v.at[2 * b] for b in range(RING)]
        dstb = [sd_v.at[2 * b + 1] for b in range(RING)]
        rows = [rows_v.at[b] for b in range(RING)]
        cid = lax.axis_index("c")
        sid = lax.axis_index("s")
        wid = sid * _NC + cid
        r0 = pl.multiple_of(sid * rpt, 8)

        def unpack(j, b):
            for i in range(B // 16):
                w = idx_v[j, pl.ds(i * 16, 16)]
                srcb[b][pl.ds(i * 16, 16)] = jnp.bitwise_and(w, 0xFFFF)
                dstb[b][pl.ds(i * 16, 16)] = jnp.right_shift(w, 16)

        pltpu.sync_copy(packed_hbm.at[wid], idx_v)
        for b in range(RING):
            unpack(b, b)
            pltpu.async_copy(g_hbm.at[srcb[b]], rows[b], sems[b])

        @pl.when(cid == 0)
        def _():
            pltpu.sync_copy(g_hbm.at[pl.ds(r0, rpt)], acc_s.at[pl.ds(r0, rpt)])

        @pl.when(cid != 0)
        def _():
            pltpu.sync_copy(zeros_hbm.at[pl.ds(r0, rpt)],
                            acc_s.at[pl.ds(r0, rpt)])

        plsc.subcore_barrier()

        def body(k, c):
            j = k * RING
            for b in range(RING):
                pltpu.make_async_copy(g_hbm.at[srcb[b]], rows[b],
                                      sems[b]).wait()
                pltpu.sync_copy(rows[b], acc_s.at[dstb[b]], add=True)
                unpack(j + b + RING, b)
                pltpu.async_copy(g_hbm.at[srcb[b]], rows[b], sems[b])
            return c
        lax.fori_loop(0, nb // RING - 1, body, 0)

        for b in range(RING):
            pltpu.make_async_copy(g_hbm.at[srcb[b]], rows[b], sems[b]).wait()
            pltpu.sync_copy(rows[b], acc_s.at[dstb[b]], add=True)

        plsc.subcore_barrier()
        pltpu.sync_copy(acc_s.at[pl.ds(r0, rpt)], out_hbm.at[cid, pl.ds(r0, rpt)])

    return scatter_kernel


_BM = 1024


def _first_body(d0_ref, d1_ref, x_ref, w_ref, dinv_ref, g_ref):
    deg = d0_ref[...] + d1_ref[...] + 1.0
    dinv = lax.rsqrt(deg)
    dinv_ref[...] = dinv
    g_ref[...] = dinv * jnp.dot(x_ref[...], w_ref[...],
                                preferred_element_type=jnp.float32)


def _mid_body(p0_ref, p1_ref, dinv_ref, b_ref, w_ref, g_ref):
    dinv = dinv_ref[...]
    h = jnp.maximum(dinv * (p0_ref[...] + p1_ref[...]) + b_ref[...], 0.0)
    g_ref[...] = dinv * jnp.dot(h, w_ref[...],
                                preferred_element_type=jnp.float32)


def _last_body(p0_ref, p1_ref, dinv_ref, b_ref, o_ref):
    o_ref[...] = dinv_ref[...] * (p0_ref[...] + p1_ref[...]) + b_ref[...]


def _row_spec(i):
    return (i, 0)


def _const_spec(i):
    return (0, 0)


def _tc_first(d0, d1, x, w, NP):
    return pl.pallas_call(
        _first_body,
        grid=(NP // _BM,),
        in_specs=[pl.BlockSpec((_BM, 1), _row_spec),
                  pl.BlockSpec((_BM, 1), _row_spec),
                  pl.BlockSpec((_BM, _D), _row_spec),
                  pl.BlockSpec((_D, _D), _const_spec)],
        out_specs=[pl.BlockSpec((_BM, 1), _row_spec),
                   pl.BlockSpec((_BM, _D), _row_spec)],
        out_shape=[jax.ShapeDtypeStruct((NP, 1), jnp.float32),
                   jax.ShapeDtypeStruct((NP, _D), jnp.float32)],
    )(d0, d1, x, w)


def _tc_mid(p0, p1, dinv, b, w, NP):
    return pl.pallas_call(
        _mid_body,
        grid=(NP // _BM,),
        in_specs=[pl.BlockSpec((_BM, _D), _row_spec),
                  pl.BlockSpec((_BM, _D), _row_spec),
                  pl.BlockSpec((_BM, 1), _row_spec),
                  pl.BlockSpec((1, _D), _const_spec),
                  pl.BlockSpec((_D, _D), _const_spec)],
        out_specs=pl.BlockSpec((_BM, _D), _row_spec),
        out_shape=jax.ShapeDtypeStruct((NP, _D), jnp.float32),
    )(p0, p1, dinv, b, w)


def _tc_last(p0, p1, dinv, b, NP):
    return pl.pallas_call(
        _last_body,
        grid=(NP // _BM,),
        in_specs=[pl.BlockSpec((_BM, _D), _row_spec),
                  pl.BlockSpec((_BM, _D), _row_spec),
                  pl.BlockSpec((_BM, 1), _row_spec),
                  pl.BlockSpec((1, _D), _const_spec)],
        out_specs=pl.BlockSpec((_BM, _D), _row_spec),
        out_shape=jax.ShapeDtypeStruct((NP, _D), jnp.float32),
    )(p0, p1, dinv, b)


def kernel(x, edge_index, W0, b0, W1, b1, W2, b2):
    N, D = x.shape
    E = edge_index.shape[1]
    NP = _pad_to(N, _BM)
    SB, SRING = 64, 3
    EP = _pad_to(E, math.lcm(_NW * _B, _NW * SB * SRING))

    xp = jnp.pad(x, ((0, NP - N), (0, 0)))
    srcf = jnp.pad(edge_index[0], (0, EP - E))
    dstf = jnp.pad(edge_index[1], (0, EP - E), constant_values=N)
    dst = dstf.reshape(_NW, EP // (_NW * _B), _B)
    packed = jnp.bitwise_or(srcf, jnp.left_shift(dstf, 16)).reshape(
        _NW, EP // (_NW * SB), SB)
    zeros = jnp.zeros((NP, _D), jnp.float32)

    deg_kernel = _make_deg_kernel(EP, NP)
    scatter_kernel = _make_scatter_kernel(EP, NP, SB, SRING)

    degp = deg_kernel(dst)
    d0 = degp[0].reshape(NP, 1)
    d1 = degp[1].reshape(NP, 1)

    dinv, g = _tc_first(d0, d1, xp, W0, NP)
    b0r = b0.reshape(1, _D)
    b1r = b1.reshape(1, _D)
    b2r = b2.reshape(1, _D)

    p = scatter_kernel(g, packed, zeros)
    g = _tc_mid(p[0], p[1], dinv, b0r, W1, NP)

    p = scatter_kernel(g, packed, zeros)
    g = _tc_mid(p[0], p[1], dinv, b1r, W2, NP)

    p = scatter_kernel(g, packed, zeros)
    out = _tc_last(p[0], p[1], dinv, b2r, NP)
    return out[:N]


# speedup vs baseline: 1.3254x; 1.3254x over previous
import functools
import math

import jax
import jax.numpy as jnp
from jax import lax
from jax.experimental import pallas as pl
from jax.experimental.pallas import tpu as pltpu
from jax.experimental.pallas import tpu_sc as plsc

_D = 128
_NC = 2
_NS = 16
_NW = _NC * _NS
_B = 128
_RING = 2


def _pad_to(n, m):
    return ((n + m - 1) // m) * m


@functools.cache
def _make_deg_kernel(EP, NP):
    nb = EP // (_NW * _B)
    rpt = NP // _NS
    mesh = plsc.VectorSubcoreMesh(core_axis_name="c", subcore_axis_name="s",
                                  num_cores=_NC, num_subcores=_NS)

    @functools.partial(
        pl.kernel,
        out_type=jax.ShapeDtypeStruct((_NC, NP), jnp.float32),
        mesh=mesh,
        scratch_types=[
            pltpu.VMEM((nb, _B), jnp.int32),
            pltpu.VMEM((_B,), jnp.float32),
            pltpu.VMEM((rpt,), jnp.float32),
            pltpu.VMEM_SHARED((NP,), jnp.float32),
        ],
    )
    def deg_kernel(dst_hbm, out_hbm, idx_v, ones_v, zeros_v, hist_s):
        cid = lax.axis_index("c")
        sid = lax.axis_index("s")
        wid = sid * _NC + cid
        pltpu.sync_copy(dst_hbm.at[wid], idx_v)
        one16 = jnp.ones((16,), jnp.float32)
        zero16 = jnp.zeros((16,), jnp.float32)
        for i in range(_B // 16):
            ones_v[pl.ds(i * 16, 16)] = one16

        def zbody(i, c):
            zeros_v[pl.ds(pl.multiple_of(i * 16, 8), 16)] = zero16
            return c
        lax.fori_loop(0, rpt // 16, zbody, 0)
        r0 = pl.multiple_of(sid * rpt, 8)
        pltpu.sync_copy(zeros_v, hist_s.at[pl.ds(r0, rpt)])
        plsc.subcore_barrier()

        def body(j, c):
            pltpu.sync_copy(ones_v, hist_s.at[idx_v.at[j]], add=True)
            return c
        lax.fori_loop(0, nb, body, 0)
        plsc.subcore_barrier()
        pltpu.sync_copy(hist_s.at[pl.ds(r0, rpt)], out_hbm.at[cid, pl.ds(r0, rpt)])

    return deg_kernel


_IR = 4
_GR = 2


@functools.cache
def _make_scatter_kernel(EP, NP, B=_B):
    nb = EP // (_NW * B)
    rpt = NP // _NS
    assert nb % _IR == 0 and nb >= 2 * _IR
    mesh = plsc.VectorSubcoreMesh(core_axis_name="c", subcore_axis_name="s",
                                  num_cores=_NC, num_subcores=_NS)

    @functools.partial(
        pl.kernel,
        out_type=jax.ShapeDtypeStruct((_NC, NP, _D), jnp.float32),
        mesh=mesh,
        scratch_types=(
            [pltpu.VMEM((_IR, 2, B), jnp.int32),
             pltpu.VMEM((_GR, B, _D), jnp.float32),
             pltpu.VMEM_SHARED((NP, _D), jnp.float32)]
            + [pltpu.SemaphoreType.DMA for _ in range(_IR + _GR)]
        ),
    )
    def scatter_kernel(g_hbm, sd_hbm, zeros_hbm, out_hbm, sd_v, rows_v,
                       acc_s, *sems):
        isem = sems[:_IR]
        gsem = sems[_IR:]
        rows = [rows_v.at[b] for b in range(_GR)]
        cid = lax.axis_index("c")
        sid = lax.axis_index("s")
        wid = sid * _NC + cid
        r0 = pl.multiple_of(sid * rpt, 8)

        for q in range(_IR):
            pltpu.async_copy(sd_hbm.at[wid, q], sd_v.at[q], isem[q])
        for j in range(_GR):
            pltpu.make_async_copy(sd_hbm.at[wid, j], sd_v.at[j],
                                  isem[j]).wait()
            pltpu.async_copy(g_hbm.at[sd_v.at[j, 0]], rows[j], gsem[j])

        @pl.when(cid == 0)
        def _():
            pltpu.sync_copy(g_hbm.at[pl.ds(r0, rpt)], acc_s.at[pl.ds(r0, rpt)])

        @pl.when(cid != 0)
        def _():
            pltpu.sync_copy(zeros_hbm.at[pl.ds(r0, rpt)],
                            acc_s.at[pl.ds(r0, rpt)])

        plsc.subcore_barrier()

        def step(j, u, refill, launch):
            b = u % _GR
            pltpu.make_async_copy(g_hbm.at[sd_v.at[u, 0]], rows[b],
                                  gsem[b]).wait()
            pltpu.sync_copy(rows[b], acc_s.at[sd_v.at[u, 1]], add=True)
            if refill:
                pltpu.async_copy(sd_hbm.at[wid, j + _IR], sd_v.at[u], isem[u])
            if launch:
                q2 = (u + _GR) % _IR
                pltpu.make_async_copy(sd_hbm.at[wid, j + _GR],
                                      sd_v.at[q2], isem[q2]).wait()
                pltpu.async_copy(g_hbm.at[sd_v.at[q2, 0]], rows[b], gsem[b])

        def body(k, c):
            jb = k * _IR
            for u in range(_IR):
                step(jb + u, u, True, True)
            return c
        lax.fori_loop(0, nb // _IR - 1, body, 0)

        jb = nb - _IR
        for u in range(_IR):
            step(jb + u, u, False, u < _IR - _GR)

        plsc.subcore_barrier()
        pltpu.sync_copy(acc_s.at[pl.ds(r0, rpt)], out_hbm.at[cid, pl.ds(r0, rpt)])

    return scatter_kernel


_BM = 1024


def _first_body(d0_ref, d1_ref, x_ref, w_ref, dinv_ref, g_ref):
    deg = d0_ref[...] + d1_ref[...] + 1.0
    dinv = lax.rsqrt(deg)
    dinv_ref[...] = dinv
    g_ref[...] = dinv * jnp.dot(x_ref[...], w_ref[...],
                                preferred_element_type=jnp.float32)


def _mid_body(p0_ref, p1_ref, dinv_ref, b_ref, w_ref, g_ref):
    dinv = dinv_ref[...]
    h = jnp.maximum(dinv * (p0_ref[...] + p1_ref[...]) + b_ref[...], 0.0)
    g_ref[...] = dinv * jnp.dot(h, w_ref[...],
                                preferred_element_type=jnp.float32)


def _last_body(p0_ref, p1_ref, dinv_ref, b_ref, o_ref):
    o_ref[...] = dinv_ref[...] * (p0_ref[...] + p1_ref[...]) + b_ref[...]


def _row_spec(i):
    return (i, 0)


def _const_spec(i):
    return (0, 0)


def _tc_first(d0, d1, x, w, NP):
    return pl.pallas_call(
        _first_body,
        grid=(NP // _BM,),
        in_specs=[pl.BlockSpec((_BM, 1), _row_spec),
                  pl.BlockSpec((_BM, 1), _row_spec),
                  pl.BlockSpec((_BM, _D), _row_spec),
                  pl.BlockSpec((_D, _D), _const_spec)],
        out_specs=[pl.BlockSpec((_BM, 1), _row_spec),
                   pl.BlockSpec((_BM, _D), _row_spec)],
        out_shape=[jax.ShapeDtypeStruct((NP, 1), jnp.float32),
                   jax.ShapeDtypeStruct((NP, _D), jnp.float32)],
    )(d0, d1, x, w)


def _tc_mid(p0, p1, dinv, b, w, NP):
    return pl.pallas_call(
        _mid_body,
        grid=(NP // _BM,),
        in_specs=[pl.BlockSpec((_BM, _D), _row_spec),
                  pl.BlockSpec((_BM, _D), _row_spec),
                  pl.BlockSpec((_BM, 1), _row_spec),
                  pl.BlockSpec((1, _D), _const_spec),
                  pl.BlockSpec((_D, _D), _const_spec)],
        out_specs=pl.BlockSpec((_BM, _D), _row_spec),
        out_shape=jax.ShapeDtypeStruct((NP, _D), jnp.float32),
    )(p0, p1, dinv, b, w)


def _tc_last(p0, p1, dinv, b, NP):
    return pl.pallas_call(
        _last_body,
        grid=(NP // _BM,),
        in_specs=[pl.BlockSpec((_BM, _D), _row_spec),
                  pl.BlockSpec((_BM, _D), _row_spec),
                  pl.BlockSpec((_BM, 1), _row_spec),
                  pl.BlockSpec((1, _D), _const_spec)],
        out_specs=pl.BlockSpec((_BM, _D), _row_spec),
        out_shape=jax.ShapeDtypeStruct((NP, _D), jnp.float32),
    )(p0, p1, dinv, b)


def kernel(x, edge_index, W0, b0, W1, b1, W2, b2):
    N, D = x.shape
    E = edge_index.shape[1]
    NP = _pad_to(N, _BM)
    EP = _pad_to(E, math.lcm(_NW * _B, _NW * _B * _IR))

    xp = jnp.pad(x, ((0, NP - N), (0, 0)))
    srcf = jnp.pad(edge_index[0], (0, EP - E))
    dstf = jnp.pad(edge_index[1], (0, EP - E), constant_values=N)
    nb = EP // (_NW * _B)
    dst = dstf.reshape(_NW, nb, _B)
    sd = jnp.stack([srcf.reshape(_NW, nb, _B), dst], axis=2)
    zeros = jnp.zeros((NP, _D), jnp.float32)

    deg_kernel = _make_deg_kernel(EP, NP)
    scatter_kernel = _make_scatter_kernel(EP, NP)

    degp = deg_kernel(dst)
    d0 = degp[0].reshape(NP, 1)
    d1 = degp[1].reshape(NP, 1)

    dinv, g = _tc_first(d0, d1, xp, W0, NP)
    b0r = b0.reshape(1, _D)
    b1r = b1.reshape(1, _D)
    b2r = b2.reshape(1, _D)

    p = scatter_kernel(g, sd, zeros)
    g = _tc_mid(p[0], p[1], dinv, b0r, W1, NP)

    p = scatter_kernel(g, sd, zeros)
    g = _tc_mid(p[0], p[1], dinv, b1r, W2, NP)

    p = scatter_kernel(g, sd, zeros)
    out = _tc_last(p[0], p[1], dinv, b2r, NP)
    return out[:N]
